# baseline (speedup 1.0000x reference)
import jax
import jax.numpy as jnp
from jax import lax
from jax.experimental import pallas as pl
from jax.experimental.pallas import tpu as pltpu

_CR = 2048


def _copy_body(x_hbm, o_hbm, v0, v1, is0, is1, os0, os1):
    nchunks = x_hbm.shape[0] // _CR
    bufs = (v0, v1)
    isems = (is0, is1)
    osems = (os0, os1)

    def start_in(b, c):
        pltpu.make_async_copy(x_hbm.at[pl.ds(c * _CR, _CR)], bufs[b], isems[b]).start()

    def wait_in(b, c):
        pltpu.make_async_copy(x_hbm.at[pl.ds(c * _CR, _CR)], bufs[b], isems[b]).wait()

    def start_out(b, c):
        pltpu.make_async_copy(bufs[b], o_hbm.at[pl.ds(c * _CR, _CR)], osems[b]).start()

    def wait_out(b, c):
        pltpu.make_async_copy(bufs[b], o_hbm.at[pl.ds(c * _CR, _CR)], osems[b]).wait()

    start_in(0, 0)
    start_in(1, 1)

    def body(p, carry):
        c0 = 2 * p
        wait_in(0, c0)
        start_out(0, c0)
        wait_in(1, c0 + 1)
        start_out(1, c0 + 1)
        wait_out(0, c0)
        start_in(0, c0 + 2)
        wait_out(1, c0 + 1)
        start_in(1, c0 + 3)
        return carry

    lax.fori_loop(0, nchunks // 2 - 1, body, 0)

    last = nchunks - 2
    wait_in(0, last)
    start_out(0, last)
    wait_in(1, last + 1)
    start_out(1, last + 1)
    wait_out(0, last)
    wait_out(1, last + 1)


def kernel(stack):
    n, s, d = stack.shape
    x = stack.transpose(0, 2, 1).reshape(n * d, s)
    rows = n * d
    y = pl.pallas_call(
        _copy_body,
        in_specs=[pl.BlockSpec(memory_space=pl.ANY)],
        out_specs=pl.BlockSpec(memory_space=pl.ANY),
        out_shape=jax.ShapeDtypeStruct((rows, s), stack.dtype),
        scratch_shapes=[
            pltpu.VMEM((_CR, 1024), jnp.float32),
            pltpu.VMEM((_CR, 1024), jnp.float32),
            pltpu.SemaphoreType.DMA,
            pltpu.SemaphoreType.DMA,
            pltpu.SemaphoreType.DMA,
            pltpu.SemaphoreType.DMA,
        ],
    )(x)
    return y.reshape(n, d, s).transpose(0, 2, 1)


# speedup vs baseline: 1.0478x; 1.0478x over previous
import jax
import jax.numpy as jnp
from jax.experimental import pallas as pl
from jax.experimental.pallas import tpu as pltpu

_CR = 2048
_NBUF = 4
_D = 2


def _copy_body(x_hbm, o_hbm, v0, v1, v2, v3, *sems):
    nchunks = x_hbm.shape[0] // _CR
    bufs = (v0, v1, v2, v3)
    isems = sems[:_NBUF]
    osems = sems[_NBUF:]

    def in_copy(c):
        b = c % _NBUF
        return pltpu.make_async_copy(
            x_hbm.at[pl.ds(c * _CR, _CR)], bufs[b], isems[b]
        )

    def out_copy(c):
        b = c % _NBUF
        return pltpu.make_async_copy(
            bufs[b], o_hbm.at[pl.ds(c * _CR, _CR)], osems[b]
        )

    for c in range(_D):
        in_copy(c).start()
    for c in range(nchunks):
        in_copy(c).wait()
        out_copy(c).start()
        if c >= _NBUF - _D:
            out_copy(c - (_NBUF - _D)).wait()
        if c + _D < nchunks:
            in_copy(c + _D).start()
    for c in range(nchunks - (_NBUF - _D), nchunks):
        out_copy(c).wait()


def kernel(stack):
    n, s, d = stack.shape
    x = stack.transpose(0, 2, 1).reshape(n * d, s)
    rows = n * d
    y = pl.pallas_call(
        _copy_body,
        in_specs=[pl.BlockSpec(memory_space=pl.ANY)],
        out_specs=pl.BlockSpec(memory_space=pl.ANY),
        out_shape=jax.ShapeDtypeStruct((rows, s), stack.dtype),
        scratch_shapes=[pltpu.VMEM((_CR, 1024), jnp.float32)] * _NBUF
        + [pltpu.SemaphoreType.DMA] * (2 * _NBUF),
    )(x)
    return y.reshape(n, d, s).transpose(0, 2, 1)
